# baseline (speedup 1.0000x reference)
import dataclasses

import jax
import jax.numpy as jnp
from jax import lax
from jax.experimental import pallas as pl
from jax.experimental.pallas import tpu as pltpu
from jax.experimental.pallas import tpu_sc as plsc

VOCAB = 1000
B, S = 1024, 50
N = B * S
NC, NS = 2, 16
NW = NC * NS
R = N // NW
L = 16
V = R // L
NB = 8
W = 128
CH = 128
CAP = 1792
SLOTS = 20


def _body(txt_hbm, tgt_hbm, vis_hbm, out_hbm,
          gbuf, tgtbuf, visbuf, rows_v, lanes_v, off_v, part_v,
          cnt_s, cs_s, cv_s, sem_a, sem_b):
    wid = lax.axis_index("s") * NC + lax.axis_index("c")
    base = wid * R

    pltpu.sync_copy(vis_hbm.at[pl.ds(base, R)], visbuf)
    pltpu.sync_copy(tgt_hbm.at[pl.ds(base, R)], tgtbuf.at[pl.ds(8, R)])

    iota = lax.iota(jnp.int32, L)
    zero_f = jnp.zeros((L,), jnp.float32)
    zero_i = jnp.zeros((L,), jnp.int32)

    off_v[pl.ds(0, L)] = iota * CAP

    def step_a(j, carry):
        a_vis, a_cnt = carry
        cur = tgtbuf[pl.ds(8 + j * L, L)]
        prev = tgtbuf[pl.ds(7 + j * L, L)]
        pos = j * L + iota
        vis_i = jnp.where(cur > VOCAB, 1, 0)
        first_i = jnp.where(pos % S == 0, 1, 0)
        prev_i = jnp.where(prev > 0, 1, 0)
        txt_i = (1 - vis_i) * jnp.minimum(first_i + prev_i, 1)
        a_cnt = a_cnt + (txt_i + vis_i).astype(jnp.float32)
        vv = visbuf[pl.ds(j * L, L)]
        a_vis = a_vis + vv * vis_i.astype(jnp.float32)
        tc = jnp.where(cur > VOCAB, 0, cur)
        gsel = txt_i * (tc // W) + (1 - txt_i) * NB
        packed = pos * W + tc % W
        sg, sp = plsc.sort_key_val(gsel, packed)
        prevg = jnp.where(
            iota == 0, -1,
            sg.at[jnp.maximum(iota - 1, 0)].get(mode="promise_in_bounds"))
        nextg = jnp.where(
            iota == L - 1, -2,
            sg.at[jnp.minimum(iota + 1, L - 1)].get(
                mode="promise_in_bounds"))
        firstidx = plsc.cummax(jnp.where(sg != prevg, iota, 0))
        rank = iota - firstidx
        pos_w = plsc.load_gather(off_v, [sg]) + rank
        plsc.store_scatter(rows_v, [pos_w], base + sp // W)
        plsc.store_scatter(lanes_v, [pos_w], sp % W)
        plsc.addupdate_scatter(off_v, [sg], rank + 1, mask=sg != nextg)
        return a_vis, a_cnt

    a_vis, a_cnt = lax.fori_loop(0, V, step_a, (zero_f, zero_f))

    offs_lo = off_v[pl.ds(0, L)]

    for t in range(NB):
        off_t = offs_lo[t]
        cnt_s[t] = off_t - t * CAP
        for v in range(CH // L):
            fill_idx = off_t + v * L + iota
            plsc.store_scatter(rows_v, [fill_idx], zero_i)
            plsc.store_scatter(lanes_v, [fill_idx], zero_i)

    for s in range(SLOTS):
        cs_s[s] = 0
        cv_s[s] = 0

    def build_txt(t, slot):
        cnt = cnt_s[t]

        def build_k(k, slot):
            live = k * CH < cnt

            @pl.when(live)
            def _():
                cs_s[slot] = t * CAP + k * CH
                cv_s[slot] = jnp.minimum(cnt - k * CH, CH)

            return slot + jnp.where(live, 1, 0)

        return lax.fori_loop(0, CAP // CH, build_k, slot)

    lax.fori_loop(0, NB, build_txt, jnp.int32(0))

    part_v[pl.ds(0, L)] = zero_f
    sems = (sem_a, sem_b)

    def extract(s):
        cv = cv_s[s]
        st = cs_s[s]
        sacc = zero_f
        for v in range(CH // L):
            p = v * L + iota
            lane16 = lanes_v[pl.ds(st + v * L, L)]
            val = plsc.load_gather(gbuf.at[s % 2], [p, lane16])
            sacc = sacc + val * jnp.where(p < cv, 1.0, 0.0)
        plsc.addupdate(part_v.at[pl.ds(0, L)], sacc)

    prev_cp = None
    for s in range(SLOTS):
        st = pl.multiple_of(cs_s[s], CH)
        cp = pltpu.make_async_copy(
            txt_hbm.at[rows_v.at[pl.ds(st, CH)],
                       pl.ds((st // CAP) * W, W)],
            gbuf.at[s % 2], sems[s % 2])
        cp.start()
        if prev_cp is not None:
            prev_cp.wait()
            extract(s - 1)
        prev_cp = cp
    prev_cp.wait()
    extract(SLOTS - 1)

    part_v[pl.ds(L, L)] = a_vis
    part_v[pl.ds(2 * L, L)] = a_cnt
    pltpu.sync_copy(part_v, out_hbm.at[pl.ds(wid * 3 * L, 3 * L)])


@jax.jit
def kernel(txt_input, vis_input, target):
    tgt_flat = target.reshape(-1)
    vis_flat = vis_input.reshape(-1)

    mesh = plsc.VectorSubcoreMesh(
        core_axis_name="c", subcore_axis_name="s",
        num_cores=NC, num_subcores=NS)
    cparams = pltpu.CompilerParams()
    if "needs_layout_passes" in pltpu.CompilerParams.__dataclass_fields__:
        cparams = dataclasses.replace(cparams, needs_layout_passes=False)
    run = pl.kernel(
        _body,
        out_type=jax.ShapeDtypeStruct((NW * 3 * L,), jnp.float32),
        mesh=mesh,
        compiler_params=cparams,
        scratch_types=[
            pltpu.VMEM((2, CH, W), jnp.float32),
            pltpu.VMEM((R + 8,), jnp.int32),
            pltpu.VMEM((R,), jnp.float32),
            pltpu.VMEM(((NB + 1) * CAP,), jnp.int32),
            pltpu.VMEM(((NB + 1) * CAP,), jnp.int32),
            pltpu.VMEM((L,), jnp.int32),
            pltpu.VMEM((3 * L,), jnp.float32),
            pltpu.SMEM((NB,), jnp.int32),
            pltpu.SMEM((SLOTS,), jnp.int32),
            pltpu.SMEM((SLOTS,), jnp.int32),
            pltpu.SemaphoreType.DMA,
            pltpu.SemaphoreType.DMA,
        ],
    )
    parts = run(txt_input, tgt_flat, vis_flat).reshape(NW, 3, L)
    sums = jnp.sum(parts, axis=(0, 2))
    return -(sums[0] + sums[1]) / sums[2]


# speedup vs baseline: 1.2353x; 1.2353x over previous
import functools

import jax
import jax.numpy as jnp
from jax import lax
from jax.experimental import pallas as pl
from jax.experimental.pallas import tpu as pltpu
from jax.experimental.pallas import tpu_sc as plsc

VOCAB = 1000
B, S = 1024, 50
N = B * S
NC, NS = 2, 16
NW = NC * NS
R = N // NW
L = 16
V = R // L
CH = 80
NCH = R // CH


def _body(txt_hbm, tgt_hbm, vis_hbm, out_hbm, tgtbuf, idx_v, gath_v, vis_v,
          part_v, gsem, vsem):
    wid = lax.axis_index("s") * NC + lax.axis_index("c")
    base = wid * R

    vis_cp = pltpu.make_async_copy(vis_hbm.at[pl.ds(base, R)], vis_v, vsem)
    vis_cp.start()
    pltpu.sync_copy(tgt_hbm.at[pl.ds(base, R)], tgtbuf.at[pl.ds(8, R)])

    @pl.loop(0, V)
    def _compute_idx(j):
        cur = tgtbuf[pl.ds(8 + j * L, L)]
        tc = jnp.where(cur > VOCAB, 0, cur)
        rows = (base + j * L) + lax.iota(jnp.int32, L)
        idx_v[pl.ds(j * L, L)] = rows * (VOCAB + 1) + tc

    gather_cps = []
    for c in range(NCH):
        cp = pltpu.make_async_copy(
            txt_hbm.at[idx_v.at[pl.ds(c * CH, CH)]],
            gath_v.at[pl.ds(c * CH, CH)], gsem)
        cp.start()
        gather_cps.append(cp)
    for cp in gather_cps:
        cp.wait()
    vis_cp.wait()

    zero = jnp.zeros((L,), jnp.float32)

    def acc_step(j, carry):
        a_txt, a_vis, a_cnt = carry
        cur = tgtbuf[pl.ds(8 + j * L, L)]
        prev = tgtbuf[pl.ds(7 + j * L, L)]
        pos = j * L + lax.iota(jnp.int32, L)
        vis_f = jnp.where(cur > VOCAB, 1.0, 0.0)
        first_f = jnp.where(pos % S == 0, 1.0, 0.0)
        prev_f = jnp.where(prev > 0, 1.0, 0.0)
        txt_f = (1.0 - vis_f) * jnp.minimum(first_f + prev_f, 1.0)
        g = gath_v[pl.ds(j * L, L)]
        vv = vis_v[pl.ds(j * L, L)]
        a_txt = a_txt + g * txt_f
        a_vis = a_vis + vv * vis_f
        a_cnt = a_cnt + txt_f + vis_f
        return a_txt, a_vis, a_cnt

    a_txt, a_vis, a_cnt = lax.fori_loop(0, V, acc_step, (zero, zero, zero))

    part_v[pl.ds(0, L)] = a_txt
    part_v[pl.ds(L, L)] = a_vis
    part_v[pl.ds(2 * L, L)] = a_cnt
    pltpu.sync_copy(part_v, out_hbm.at[pl.ds(wid * 3 * L, 3 * L)])


@jax.jit
def kernel(txt_input, vis_input, target):
    txt_flat = txt_input.reshape(-1)
    vis_flat = vis_input.reshape(-1)
    tgt_flat = target.reshape(-1)

    mesh = plsc.VectorSubcoreMesh(
        core_axis_name="c", subcore_axis_name="s",
        num_cores=NC, num_subcores=NS)
    run = pl.kernel(
        _body,
        out_type=jax.ShapeDtypeStruct((NW * 3 * L,), jnp.float32),
        mesh=mesh,
        scratch_types=[
            pltpu.VMEM((R + 8,), jnp.int32),
            pltpu.VMEM((R,), jnp.int32),
            pltpu.VMEM((R,), jnp.float32),
            pltpu.VMEM((R,), jnp.float32),
            pltpu.VMEM((3 * L,), jnp.float32),
            pltpu.SemaphoreType.DMA,
            pltpu.SemaphoreType.DMA,
        ],
    )
    parts = run(txt_flat, tgt_flat, vis_flat).reshape(NW, 3, L)
    sums = jnp.sum(parts, axis=(0, 2))
    return -(sums[0] + sums[1]) / sums[2]
